# baseline (speedup 1.0000x reference)
import functools

import numpy as np
import jax
import jax.numpy as jnp
from jax import lax
from jax.experimental import pallas as pl
from jax.experimental.pallas import tpu as pltpu
from jax.experimental.pallas import tpu_sc as plsc

N = 10000
E = 160000
B = 1024

_info = plsc.get_sparse_core_info()
NC = _info.num_cores
NS = _info.num_subcores
L = _info.num_lanes
NW = NC * NS
EPW = E // NW
C = 40
NCHUNK = EPW // C

NB = 1024
NBLK = (N + NB - 1) // NB


EB = E // NBLK


def _tc_prep_body(x_ref, h_ref, j_ref, xt_ref, xh_ref, j16_ref):
    pid = pl.program_id(0)
    col = pid * NB + lax.broadcasted_iota(jnp.int32, (1, NB), 1)
    valid = col < N
    xb = jnp.where(valid, x_ref[...], 0.0)
    hb = jnp.where(valid, h_ref[...], 0.0)
    xt = xb.T
    xt_ref[...] = xt.astype(jnp.bfloat16)

    @pl.when(pid == 0)
    def _():
        xh_ref[...] = jnp.zeros_like(xh_ref)

    xh_ref[...] += lax.dot_general(
        hb, xt, (((1,), (0,)), ((), ())),
        preferred_element_type=jnp.float32)

    j16_ref[...] = jnp.broadcast_to(j_ref[...].reshape(EB, 1), (EB, 16))


_tc_prep = pl.pallas_call(
    _tc_prep_body,
    grid=(NBLK,),
    in_specs=[
        pl.BlockSpec((B, NB), lambda i: (0, i)),
        pl.BlockSpec((1, NB), lambda i: (0, i)),
        pl.BlockSpec((1, EB), lambda i: (0, i)),
    ],
    out_specs=[
        pl.BlockSpec((NB, B), lambda i: (i, 0)),
        pl.BlockSpec((1, B), lambda i: (0, 0)),
        pl.BlockSpec((EB, 16), lambda i: (i, 0)),
    ],
    out_shape=[
        jax.ShapeDtypeStruct((N, B), jnp.bfloat16),
        jax.ShapeDtypeStruct((1, B), jnp.float32),
        jax.ShapeDtypeStruct((E, 16), jnp.float32),
    ],
)


R = 8
S = 5
AHEAD = S - 1
NSUB = EPW // R
NOUT = NSUB // S

QG = 16
NG = B // (QG * L)


@functools.partial(
    pl.kernel,
    mesh=plsc.VectorSubcoreMesh(core_axis_name="c", subcore_axis_name="s"),
    out_type=jax.ShapeDtypeStruct((NW, B), jnp.float32),
    scratch_types=[
        pltpu.VMEM((EPW,), jnp.int32),
        pltpu.VMEM((EPW,), jnp.int32),
        pltpu.VMEM((S, R, 16), jnp.float32),
        pltpu.VMEM((S, R, B // 2), jnp.int32),
        pltpu.VMEM((S, R, B // 2), jnp.int32),
        pltpu.VMEM((B,), jnp.float32),
        pltpu.SemaphoreType.DMA,
        pltpu.SemaphoreType.DMA,
        pltpu.SemaphoreType.DMA,
        pltpu.SemaphoreType.DMA,
        pltpu.SemaphoreType.DMA,
    ],
)
def _sc_edges(xt_hbm, j16_hbm, ei_hbm, ej_hbm, out_hbm,
              ei_v, ej_v, jv2, xi_v, xj_v, acc, *sems):
    wid = lax.axis_index("s") * NC + lax.axis_index("c")
    base = wid * EPW

    zero = jnp.zeros((L,), jnp.float32)
    for q in range(B // L):
        acc[pl.ds(q * L, L)] = zero

    cpi = pltpu.async_copy(ei_hbm.at[pl.ds(base, EPW)], ei_v, sems[0])
    cpj = pltpu.async_copy(ej_hbm.at[pl.ds(base, EPW)], ej_v, sems[1])
    cpi.wait()
    cpj.wait()

    def fire(sub, slot):
        pltpu.async_copy(xt_hbm.at[ei_v.at[pl.ds(sub * R, R)]],
                         xi_v.at[slot], sems[slot])
        pltpu.async_copy(xt_hbm.at[ej_v.at[pl.ds(sub * R, R)]],
                         xj_v.at[slot], sems[slot])
        pltpu.async_copy(j16_hbm.at[pl.ds(base + sub * R, R)],
                         jv2.at[slot], sems[slot])

    def drain(slot):
        pltpu.make_async_copy(xt_hbm.at[pl.ds(0, R)], xi_v.at[slot],
                              sems[slot]).wait()
        pltpu.make_async_copy(xt_hbm.at[pl.ds(0, R)], xj_v.at[slot],
                              sems[slot]).wait()
        pltpu.make_async_copy(j16_hbm.at[pl.ds(0, R)], jv2.at[slot],
                              sems[slot]).wait()

    for k in range(AHEAD):
        fire(k, k)

    def outer_body(o, carry):
        for k in range(S):
            s = o * S + k
            drain(k)
            @pl.when(s + AHEAD < NSUB)
            def _():
                fire(s + AHEAD, (k + AHEAD) % S)

            for g in range(NG):
                bq = g * QG * L

                hi_mask = jnp.full((L,), -65536, jnp.int32)

                def edge_body(e, accs, _bq=bq // 2, _k=k):
                    jb = jv2[_k, e, :]
                    out = []
                    for u in range(QG // 2):
                        ri = xi_v[_k, e, pl.ds(_bq + u * L, L)]
                        rj = xj_v[_k, e, pl.ds(_bq + u * L, L)]
                        lo_i = lax.bitcast_convert_type(lax.shift_left(ri, 16), jnp.float32)
                        lo_j = lax.bitcast_convert_type(lax.shift_left(rj, 16), jnp.float32)
                        hi_i = lax.bitcast_convert_type(ri & hi_mask, jnp.float32)
                        hi_j = lax.bitcast_convert_type(rj & hi_mask, jnp.float32)
                        out.append(accs[2 * u] + lo_i * lo_j * jb)
                        out.append(accs[2 * u + 1] + hi_i * hi_j * jb)
                    return tuple(out)

                init = tuple(acc[pl.ds(bq + q * L, L)] for q in range(QG))
                accs = lax.fori_loop(0, R, edge_body, init)
                for q in range(QG):
                    acc[pl.ds(bq + q * L, L)] = accs[q]
        return carry

    lax.fori_loop(0, NOUT, outer_body, 0)
    pltpu.sync_copy(acc, out_hbm.at[wid])


def _perm_matrix():
    p = np.arange(B)
    m, t = p // 32, p % 32
    nat = np.where(t < 16, 32 * m + 2 * t, 32 * m + 2 * (t - 16) + 1)
    mat = np.zeros((B, B), np.float32)
    mat[p, nat] = 1.0
    return jnp.asarray(mat)


def _tc_combine_body(parts_ref, xh_ref, perm_ref, out_ref):
    psum = jnp.sum(parts_ref[...], axis=0, keepdims=True)
    out_ref[...] = xh_ref[...] + lax.dot_general(
        psum, perm_ref[...], (((1,), (0,)), ((), ())),
        preferred_element_type=jnp.float32)


_tc_combine = pl.pallas_call(
    _tc_combine_body,
    out_shape=jax.ShapeDtypeStruct((1, B), jnp.float32),
)


def kernel(x, h, J, edge_idx_i, edge_idx_j):
    xt, xh, j16 = _tc_prep(x, h.reshape(1, N), J.reshape(1, E))
    xt32 = lax.bitcast_convert_type(xt.reshape(N, B // 2, 2), jnp.int32)
    parts = _sc_edges(xt32, j16, edge_idx_i, edge_idx_j)
    out = _tc_combine(parts, xh, _perm_matrix())
    return out.reshape(B)


# speedup vs baseline: 1.0678x; 1.0678x over previous
import functools

import numpy as np
import jax
import jax.numpy as jnp
from jax import lax
from jax.experimental import pallas as pl
from jax.experimental.pallas import tpu as pltpu
from jax.experimental.pallas import tpu_sc as plsc

N = 10000
E = 160000
B = 1024

_info = plsc.get_sparse_core_info()
NC = _info.num_cores
NS = _info.num_subcores
L = _info.num_lanes
NW = NC * NS
EPW = E // NW
C = 40
NCHUNK = EPW // C

NB = 1024
NBLK = (N + NB - 1) // NB


EB = E // NBLK


def _tc_prep_body(x_ref, h_ref, j_ref, xt_ref, xh_ref, j16_ref):
    pid = pl.program_id(0)
    col = pid * NB + lax.broadcasted_iota(jnp.int32, (1, NB), 1)
    valid = col < N
    xb = jnp.where(valid, x_ref[...], 0.0)
    hb = jnp.where(valid, h_ref[...], 0.0)
    xt = xb.T
    xt_ref[...] = xt.astype(jnp.bfloat16)

    @pl.when(pid == 0)
    def _():
        xh_ref[...] = jnp.zeros_like(xh_ref)

    xh_ref[...] += lax.dot_general(
        hb, xt, (((1,), (0,)), ((), ())),
        preferred_element_type=jnp.float32)

    j16_ref[...] = jnp.broadcast_to(j_ref[...].reshape(EB, 1), (EB, 16))


_tc_prep = pl.pallas_call(
    _tc_prep_body,
    grid=(NBLK,),
    in_specs=[
        pl.BlockSpec((B, NB), lambda i: (0, i)),
        pl.BlockSpec((1, NB), lambda i: (0, i)),
        pl.BlockSpec((1, EB), lambda i: (0, i)),
    ],
    out_specs=[
        pl.BlockSpec((NB, B), lambda i: (i, 0)),
        pl.BlockSpec((1, B), lambda i: (0, 0)),
        pl.BlockSpec((EB, 16), lambda i: (i, 0)),
    ],
    out_shape=[
        jax.ShapeDtypeStruct((N, B), jnp.bfloat16),
        jax.ShapeDtypeStruct((1, B), jnp.float32),
        jax.ShapeDtypeStruct((E, 16), jnp.float32),
    ],
)


R = 40
S = 2
NSUB = EPW // R
B2 = B // 2

QG = 16
NG = B // (QG * L)


@functools.partial(
    pl.kernel,
    mesh=plsc.VectorSubcoreMesh(core_axis_name="c", subcore_axis_name="s"),
    out_type=jax.ShapeDtypeStruct((NW, B), jnp.float32),
    scratch_types=[
        pltpu.VMEM((2 * EPW,), jnp.int32),
        pltpu.VMEM((S, R, 16), jnp.float32),
        pltpu.VMEM((S, 2 * R, B2), jnp.int32),
        pltpu.VMEM((B,), jnp.float32),
        pltpu.SemaphoreType.DMA,
        pltpu.SemaphoreType.DMA,
        pltpu.SemaphoreType.DMA,
    ],
)
def _sc_edges(xt_hbm, j16_hbm, eij_hbm, out_hbm,
              eij_v, jv2, xij_v, acc, *sems):
    wid = lax.axis_index("s") * NC + lax.axis_index("c")
    base = wid * EPW

    zero = jnp.zeros((L,), jnp.float32)
    for q in range(B // L):
        acc[pl.ds(q * L, L)] = zero

    pltpu.async_copy(eij_hbm.at[pl.ds(2 * base, 2 * EPW)], eij_v,
                     sems[2]).wait()

    def fire(sub, slot):
        pltpu.async_copy(xt_hbm.at[eij_v.at[pl.ds(sub * 2 * R, 2 * R)]],
                         xij_v.at[slot], sems[slot])
        pltpu.async_copy(j16_hbm.at[pl.ds(base + sub * R, R)],
                         jv2.at[slot], sems[slot])

    def drain(slot):
        pltpu.make_async_copy(xt_hbm.at[pl.ds(0, 2 * R)], xij_v.at[slot],
                              sems[slot]).wait()
        pltpu.make_async_copy(j16_hbm.at[pl.ds(0, R)], jv2.at[slot],
                              sems[slot]).wait()

    hi_mask = jnp.full((L,), -65536, jnp.int32)

    def compute(k):
        for g in range(NG):
            bq = g * QG * L

            def edge_body(e, accs, _bq=bq // 2, _k=k):
                jb = jv2[_k, e, :]
                out = []
                for u in range(QG // 2):
                    ri = xij_v[_k, 2 * e, pl.ds(_bq + u * L, L)]
                    rj = xij_v[_k, 2 * e + 1, pl.ds(_bq + u * L, L)]
                    lo_i = lax.bitcast_convert_type(
                        lax.shift_left(ri, 16), jnp.float32)
                    lo_j = lax.bitcast_convert_type(
                        lax.shift_left(rj, 16), jnp.float32)
                    hi_i = lax.bitcast_convert_type(ri & hi_mask, jnp.float32)
                    hi_j = lax.bitcast_convert_type(rj & hi_mask, jnp.float32)
                    out.append(accs[2 * u] + lo_i * lo_j * jb)
                    out.append(accs[2 * u + 1] + hi_i * hi_j * jb)
                return tuple(out)

            init = tuple(acc[pl.ds(bq + q * L, L)] for q in range(QG))
            accs = lax.fori_loop(0, R, edge_body, init)
            for q in range(QG):
                acc[pl.ds(bq + q * L, L)] = accs[q]

    def step(s, slot):
        drain(slot)

        @pl.when(s + 1 < NSUB)
        def _():
            fire(s + 1, 1 - slot)

        compute(slot)

    fire(0, 0)

    def outer_body(o, carry):
        step(2 * o, 0)
        step(2 * o + 1, 1)
        return carry

    lax.fori_loop(0, NSUB // 2, outer_body, 0)
    step(NSUB - 1, 0)
    pltpu.sync_copy(acc, out_hbm.at[wid])


def _perm_matrix():
    p = np.arange(B)
    m, t = p // 32, p % 32
    nat = np.where(t < 16, 32 * m + 2 * t, 32 * m + 2 * (t - 16) + 1)
    mat = np.zeros((B, B), np.float32)
    mat[p, nat] = 1.0
    return jnp.asarray(mat)


def _tc_combine_body(parts_ref, xh_ref, perm_ref, out_ref):
    psum = jnp.sum(parts_ref[...], axis=0, keepdims=True)
    out_ref[...] = xh_ref[...] + lax.dot_general(
        psum, perm_ref[...], (((1,), (0,)), ((), ())),
        preferred_element_type=jnp.float32)


_tc_combine = pl.pallas_call(
    _tc_combine_body,
    out_shape=jax.ShapeDtypeStruct((1, B), jnp.float32),
)


def kernel(x, h, J, edge_idx_i, edge_idx_j):
    xt, xh, j16 = _tc_prep(x, h.reshape(1, N), J.reshape(1, E))
    xt32 = lax.bitcast_convert_type(xt.reshape(N, B // 2, 2), jnp.int32)
    eij = jnp.stack([edge_idx_i, edge_idx_j], axis=1).reshape(2 * E)
    parts = _sc_edges(xt32, j16, eij)
    out = _tc_combine(parts, xh, _perm_matrix())
    return out.reshape(B)
